# speedup vs baseline: 19.9853x; 19.9853x over previous
import functools

import jax
import jax.numpy as jnp
from jax.experimental import pallas as pl

NUM_VARS = 512
HIDDEN = 16
TOPK = 8
TEMP = 1.0


def _routing_kernel(ve_ref, wq_ref, bq_ref, wk_ref, bk_ref, s_ref):
    ve = ve_ref[...]
    q = jax.lax.dot_general(ve, wq_ref[...], (((1,), (1,)), ((), ())),
                            preferred_element_type=jnp.float32) + bq_ref[...]
    k = jax.lax.dot_general(ve, wk_ref[...], (((1,), (1,)), ((), ())),
                            preferred_element_type=jnp.float32) + bk_ref[...]
    sim = jax.lax.dot_general(q, k, (((1,), (1,)), ((), ())),
                              preferred_element_type=jnp.float32)
    n = sim.shape[0]
    row = jax.lax.broadcasted_iota(jnp.int32, (n, n), 0)
    col = jax.lax.broadcasted_iota(jnp.int32, (n, n), 1)
    sim = jnp.where(row == col, jnp.float32(-1e9), sim)

    cur = sim
    s_acc = jnp.zeros_like(sim)
    denom = jnp.zeros((n, 1), jnp.float32)
    m0 = None
    for step in range(TOPK):
        m = jnp.max(cur, axis=1, keepdims=True)
        if step == 0:
            m0 = m
        sel = cur == m
        first_col = jnp.min(jnp.where(sel, col, n), axis=1, keepdims=True)
        sel1 = sel & (col == first_col)
        w = jnp.exp((m - m0) * jnp.float32(1.0 / TEMP))
        s_acc = s_acc + jnp.where(sel1, w, jnp.float32(0.0))
        denom = denom + w
        cur = jnp.where(sel1, jnp.float32(-3e38), cur)
    s_ref[...] = s_acc / denom


def _combine_kernel(x_ref, s_ref, o_ref):
    o_ref[...] = jax.lax.dot_general(
        x_ref[...], s_ref[...], (((1,), (1,)), ((), ())),
        preferred_element_type=jnp.float32)


@jax.jit
def kernel(x, var_embed, Wq, bq, Wk, bk):
    Bsz, L, N = x.shape
    ve = var_embed.reshape(N, HIDDEN)

    s = pl.pallas_call(
        _routing_kernel,
        out_shape=jax.ShapeDtypeStruct((N, N), jnp.float32),
    )(ve, Wq, bq.reshape(1, HIDDEN), Wk, bk.reshape(1, HIDDEN))

    xs = x.reshape(Bsz * L, N)
    BL = 1024
    grid = (Bsz * L) // BL
    out = pl.pallas_call(
        _combine_kernel,
        grid=(grid,),
        in_specs=[
            pl.BlockSpec((BL, N), lambda i: (i, 0)),
            pl.BlockSpec((N, N), lambda i: (0, 0)),
        ],
        out_specs=pl.BlockSpec((BL, N), lambda i: (i, 0)),
        out_shape=jax.ShapeDtypeStruct((Bsz * L, N), jnp.float32),
    )(xs, s)
    return out.reshape(Bsz, L, N)


# speedup vs baseline: 22.1492x; 1.1083x over previous
import functools

import jax
import jax.numpy as jnp
from jax.experimental import pallas as pl
from jax.experimental.pallas import tpu as pltpu

NUM_VARS = 512
HIDDEN = 16
TOPK = 8
TEMP = 1.0


def _compute_s(ve_ref, wq_ref, bq_ref, wk_ref, bk_ref):
    ve = ve_ref[...]
    q = jax.lax.dot_general(ve, wq_ref[...], (((1,), (1,)), ((), ())),
                            preferred_element_type=jnp.float32) + bq_ref[...]
    k = jax.lax.dot_general(ve, wk_ref[...], (((1,), (1,)), ((), ())),
                            preferred_element_type=jnp.float32) + bk_ref[...]
    sim = jax.lax.dot_general(q, k, (((1,), (1,)), ((), ())),
                              preferred_element_type=jnp.float32)
    n = sim.shape[0]
    row = jax.lax.broadcasted_iota(jnp.int32, (n, n), 0)
    col = jax.lax.broadcasted_iota(jnp.int32, (n, n), 1)
    sim = jnp.where(row == col, jnp.float32(-1e9), sim)

    cur = sim
    s_acc = jnp.zeros_like(sim)
    denom = jnp.zeros((n, 1), jnp.float32)
    m0 = None
    for step in range(TOPK):
        m = jnp.max(cur, axis=1, keepdims=True)
        if step == 0:
            m0 = m
        sel = cur == m
        first_col = jnp.min(jnp.where(sel, col, n), axis=1, keepdims=True)
        sel1 = sel & (col == first_col)
        w = jnp.exp((m - m0) * jnp.float32(1.0 / TEMP))
        s_acc = s_acc + jnp.where(sel1, w, jnp.float32(0.0))
        denom = denom + w
        cur = jnp.where(sel1, jnp.float32(-3e38), cur)
    return s_acc / denom


def _fused_kernel(ve_ref, wq_ref, bq_ref, wk_ref, bk_ref, x_ref, o_ref, s_scr):
    @pl.when(pl.program_id(0) == 0)
    def _():
        s_scr[...] = _compute_s(ve_ref, wq_ref, bq_ref, wk_ref, bk_ref)

    o_ref[...] = jax.lax.dot_general(
        x_ref[...], s_scr[...], (((1,), (1,)), ((), ())),
        preferred_element_type=jnp.float32)


@jax.jit
def kernel(x, var_embed, Wq, bq, Wk, bk):
    Bsz, L, N = x.shape
    ve = var_embed.reshape(N, HIDDEN)

    xs = x.reshape(Bsz * L, N)
    BL = 1024
    grid = (Bsz * L) // BL
    out = pl.pallas_call(
        _fused_kernel,
        grid=(grid,),
        in_specs=[
            pl.BlockSpec((N, HIDDEN), lambda i: (0, 0)),
            pl.BlockSpec((HIDDEN, HIDDEN), lambda i: (0, 0)),
            pl.BlockSpec((1, HIDDEN), lambda i: (0, 0)),
            pl.BlockSpec((HIDDEN, HIDDEN), lambda i: (0, 0)),
            pl.BlockSpec((1, HIDDEN), lambda i: (0, 0)),
            pl.BlockSpec((BL, N), lambda i: (i, 0)),
        ],
        out_specs=pl.BlockSpec((BL, N), lambda i: (i, 0)),
        out_shape=jax.ShapeDtypeStruct((Bsz * L, N), jnp.float32),
        scratch_shapes=[pltpu.VMEM((N, N), jnp.float32)],
    )(ve, Wq, bq.reshape(1, HIDDEN), Wk, bk.reshape(1, HIDDEN), xs)
    return out.reshape(Bsz, L, N)


# speedup vs baseline: 23.8339x; 1.0761x over previous
import functools

import jax
import jax.numpy as jnp
from jax.experimental import pallas as pl
from jax.experimental.pallas import tpu as pltpu

NUM_VARS = 512
HIDDEN = 16
TOPK = 8
TEMP = 1.0


def _compute_s(ve_ref, wq_ref, bq_ref, wk_ref, bk_ref):
    ve = ve_ref[...]
    q = jax.lax.dot_general(ve, wq_ref[...], (((1,), (1,)), ((), ())),
                            preferred_element_type=jnp.float32) + bq_ref[...]
    k = jax.lax.dot_general(ve, wk_ref[...], (((1,), (1,)), ((), ())),
                            preferred_element_type=jnp.float32) + bk_ref[...]
    sim = jax.lax.dot_general(q, k, (((1,), (1,)), ((), ())),
                              preferred_element_type=jnp.float32)
    n = sim.shape[0]
    row = jax.lax.broadcasted_iota(jnp.int32, (n, n), 0)
    col = jax.lax.broadcasted_iota(jnp.int32, (n, n), 1)
    sim = jnp.where(row == col, jnp.float32(-1e9), sim)

    cur = sim
    s_acc = jnp.zeros_like(sim)
    denom = jnp.zeros((n, 1), jnp.float32)
    m0 = None
    for step in range(TOPK):
        m = jnp.max(cur, axis=1, keepdims=True)
        if step == 0:
            m0 = m
        sel = cur == m
        first_col = jnp.min(jnp.where(sel, col, n), axis=1, keepdims=True)
        sel1 = sel & (col == first_col)
        w = jnp.exp((m - m0) * jnp.float32(1.0 / TEMP))
        s_acc = s_acc + jnp.where(sel1, w, jnp.float32(0.0))
        denom = denom + w
        cur = jnp.where(sel1, jnp.float32(-3e38), cur)
    return s_acc / denom


def _fused_kernel(ve_ref, wq_ref, bq_ref, wk_ref, bk_ref, x_ref, o_ref, s_scr):
    @pl.when(pl.program_id(0) == 0)
    def _():
        s_scr[...] = _compute_s(ve_ref, wq_ref, bq_ref, wk_ref, bk_ref)

    o_ref[...] = jax.lax.dot_general(
        x_ref[...], s_scr[...], (((1,), (1,)), ((), ())),
        preferred_element_type=jnp.float32)


@jax.jit
def kernel(x, var_embed, Wq, bq, Wk, bk):
    Bsz, L, N = x.shape
    ve = var_embed.reshape(N, HIDDEN)

    xs = x.reshape(Bsz * L, N)
    BL = 2048
    grid = (Bsz * L) // BL
    out = pl.pallas_call(
        _fused_kernel,
        grid=(grid,),
        in_specs=[
            pl.BlockSpec((N, HIDDEN), lambda i: (0, 0)),
            pl.BlockSpec((HIDDEN, HIDDEN), lambda i: (0, 0)),
            pl.BlockSpec((1, HIDDEN), lambda i: (0, 0)),
            pl.BlockSpec((HIDDEN, HIDDEN), lambda i: (0, 0)),
            pl.BlockSpec((1, HIDDEN), lambda i: (0, 0)),
            pl.BlockSpec((BL, N), lambda i: (i, 0)),
        ],
        out_specs=pl.BlockSpec((BL, N), lambda i: (i, 0)),
        out_shape=jax.ShapeDtypeStruct((Bsz * L, N), jnp.float32),
        scratch_shapes=[pltpu.VMEM((N, N), jnp.float32)],
    )(ve, Wq, bq.reshape(1, HIDDEN), Wk, bk.reshape(1, HIDDEN), xs)
    return out.reshape(Bsz, L, N)


# speedup vs baseline: 26.2172x; 1.1000x over previous
import functools

import jax
import jax.numpy as jnp
from jax.experimental import pallas as pl
from jax.experimental.pallas import tpu as pltpu

NUM_VARS = 512
HIDDEN = 16
TOPK = 8
TEMP = 1.0


def _compute_s(ve_ref, wq_ref, bq_ref, wk_ref, bk_ref):
    ve = ve_ref[...]
    q = jax.lax.dot_general(ve, wq_ref[...], (((1,), (1,)), ((), ())),
                            preferred_element_type=jnp.float32) + bq_ref[...]
    k = jax.lax.dot_general(ve, wk_ref[...], (((1,), (1,)), ((), ())),
                            preferred_element_type=jnp.float32) + bk_ref[...]
    sim = jax.lax.dot_general(q, k, (((1,), (1,)), ((), ())),
                              preferred_element_type=jnp.float32)
    n = sim.shape[0]
    row = jax.lax.broadcasted_iota(jnp.int32, (n, n), 0)
    col = jax.lax.broadcasted_iota(jnp.int32, (n, n), 1)
    sim = jnp.where(row == col, jnp.float32(-1e9), sim)

    cur = sim
    s_acc = jnp.zeros_like(sim)
    denom = jnp.zeros((n, 1), jnp.float32)
    m0 = None
    for step in range(TOPK):
        m = jnp.max(cur, axis=1, keepdims=True)
        if step == 0:
            m0 = m
        sel = cur == m
        first_col = jnp.min(jnp.where(sel, col, n), axis=1, keepdims=True)
        sel1 = sel & (col == first_col)
        w = jnp.exp((m - m0) * jnp.float32(1.0 / TEMP))
        s_acc = s_acc + jnp.where(sel1, w, jnp.float32(0.0))
        denom = denom + w
        cur = jnp.where(sel1, jnp.float32(-3e38), cur)
    return s_acc / denom


def _fused_kernel(ve_ref, wq_ref, bq_ref, wk_ref, bk_ref, x_ref, o_ref, s_scr):
    @pl.when(pl.program_id(0) == 0)
    def _():
        s_scr[...] = _compute_s(ve_ref, wq_ref, bq_ref, wk_ref, bk_ref)

    o_ref[...] = jax.lax.dot_general(
        x_ref[...], s_scr[...], (((1,), (1,)), ((), ())),
        preferred_element_type=jnp.float32)


@jax.jit
def kernel(x, var_embed, Wq, bq, Wk, bk):
    Bsz, L, N = x.shape
    ve = var_embed.reshape(N, HIDDEN)

    xs = x.reshape(Bsz * L, N)
    BL = 4096
    grid = (Bsz * L) // BL
    out = pl.pallas_call(
        _fused_kernel,
        grid=(grid,),
        in_specs=[
            pl.BlockSpec((N, HIDDEN), lambda i: (0, 0)),
            pl.BlockSpec((HIDDEN, HIDDEN), lambda i: (0, 0)),
            pl.BlockSpec((1, HIDDEN), lambda i: (0, 0)),
            pl.BlockSpec((HIDDEN, HIDDEN), lambda i: (0, 0)),
            pl.BlockSpec((1, HIDDEN), lambda i: (0, 0)),
            pl.BlockSpec((BL, N), lambda i: (i, 0)),
        ],
        out_specs=pl.BlockSpec((BL, N), lambda i: (i, 0)),
        out_shape=jax.ShapeDtypeStruct((Bsz * L, N), jnp.float32),
        scratch_shapes=[pltpu.VMEM((N, N), jnp.float32)],
    )(ve, Wq, bq.reshape(1, HIDDEN), Wk, bk.reshape(1, HIDDEN), xs)
    return out.reshape(Bsz, L, N)
